# baseline (speedup 1.0000x reference)
import functools

import jax
import jax.numpy as jnp
from jax import lax
from jax.experimental import pallas as pl
from jax.experimental.pallas import tpu as pltpu
from jax.experimental.pallas import tpu_sc as plsc

D = 64
EPS = 1e-5
BBLK = 128
NBUF = 4
LANES = 16
NVREG = D // LANES

_info = plsc.get_sparse_core_info()
_NC, _NS = _info.num_cores, _info.num_subcores
_NW = _NC * _NS

_GATHER_DNUMS = lax.GatherDimensionNumbers(
    offset_dims=(), collapsed_slice_dims=(0,), start_index_map=(0,))


def _lane_perm(v, idx):
    return lax.gather(v, idx[:, None], _GATHER_DNUMS, (1,),
                      mode=lax.GatherScatterMode.PROMISE_IN_BOUNDS)


def _allreduce_sum(v):
    for sh in (8, 4, 2, 1):
        idx = lax.iota(jnp.int32, LANES) ^ sh
        v = v + _lane_perm(v, idx)
    return v


def _rsqrt_vec(v):
    iv = lax.bitcast_convert_type(v, jnp.int32)
    seed = jnp.full((LANES,), 0x5F3759DF, jnp.int32) - lax.shift_right_logical(iv, 1)
    y = lax.bitcast_convert_type(seed, jnp.float32)
    half = v * 0.5
    for _ in range(2):
        y = y * (1.5 - half * y * y)
    return y


@functools.lru_cache(maxsize=None)
def _make_sc_kernel(B, L):
    n_chunks = L
    mesh = plsc.VectorSubcoreMesh(core_axis_name="c", subcore_axis_name="s")

    @functools.partial(
        pl.kernel,
        out_type=jax.ShapeDtypeStruct((L, D, B), jnp.float32),
        mesh=mesh,
        compiler_params=pltpu.CompilerParams(
            use_tc_tiling_on_sc=True, needs_layout_passes=False),
        scratch_types=[
            pltpu.VMEM((L, BBLK), jnp.int32),
            pltpu.VMEM((NBUF, BBLK), jnp.int32),
            pltpu.VMEM((NBUF, BBLK, 2 * D), jnp.float32),
            pltpu.VMEM((NBUF, D, BBLK), jnp.float32),
            pltpu.VMEM((D,), jnp.float32),
            pltpu.VMEM((D,), jnp.float32),
            pltpu.SemaphoreType.DMA((NBUF,)),
            pltpu.SemaphoreType.DMA((NBUF,)),
        ],
    )
    def k(idsT_hbm, table_hbm, gamma_hbm, beta_hbm, out_hbm,
          idx_v, pair_v, rows_v, outt_v, g_v, b_v, gsem, osem):
        wid = lax.axis_index("s") * _NC + lax.axis_index("c")
        bcol = wid * BBLK
        pltpu.sync_copy(idsT_hbm.at[:, pl.ds(bcol, BBLK)], idx_v)
        pltpu.sync_copy(gamma_hbm, g_v)
        pltpu.sync_copy(beta_hbm, b_v)
        g = [g_v[pl.ds(LANES * t, LANES)] for t in range(NVREG)]
        b = [b_v[pl.ds(LANES * t, LANES)] for t in range(NVREG)]
        lane_iota = lax.iota(jnp.int32, LANES)
        dim_idx = [lane_iota + LANES * t for t in range(NVREG)]

        def start_gather(j, buf):
            for t in range(BBLK // LANES):
                ids16 = idx_v[j, pl.ds(LANES * t, LANES)]
                pair_v[buf, pl.ds(LANES * t, LANES)] = (
                    lax.shift_right_logical(ids16, 1))
            pltpu.async_copy(table_hbm.at[pair_v.at[buf]], rows_v.at[buf],
                             gsem.at[buf])

        start_gather(0, 0)
        start_gather(1, 1)

        def ln_token(buf, i, tok):
            col = (tok & 1) * D
            x = [rows_v[buf, i, pl.ds(col + LANES * t, LANES)]
                 for t in range(NVREG)]
            s = (x[0] + x[1]) + (x[2] + x[3])
            q = (x[0] * x[0] + x[1] * x[1]) + (x[2] * x[2] + x[3] * x[3])
            mv = _allreduce_sum(s) * (1.0 / D)
            var = _allreduce_sum(q) * (1.0 / D) - mv * mv
            rv = _rsqrt_vec(var + EPS)
            iv = jnp.full((LANES,), i, jnp.int32)
            for t in range(NVREG):
                a = rv * g[t]
                plsc.store_scatter(outt_v.at[buf], [dim_idx[t], iv],
                                   (x[t] - mv) * a + b[t])

        def chunk_body(j, carry):
            buf = lax.rem(j, NBUF)
            buf2 = lax.rem(j + 2, NBUF)

            @pl.when(j + 2 < n_chunks)
            def _():
                start_gather(j + 2, buf2)

            pltpu.make_async_copy(
                table_hbm.at[pair_v.at[buf]], rows_v.at[buf],
                gsem.at[buf]).wait()

            @pl.when(j >= NBUF)
            def _():
                pltpu.make_async_copy(
                    outt_v.at[buf],
                    out_hbm.at[j - NBUF, :, pl.ds(bcol, BBLK)],
                    osem.at[buf]).wait()

            def row_body(gi, c2):
                ids16 = idx_v[j, pl.ds(gi * LANES, LANES)]
                for u in range(LANES):
                    ln_token(buf, gi * LANES + u, ids16[u])
                return c2

            lax.fori_loop(0, BBLK // LANES, row_body, 0, unroll=1)
            pltpu.async_copy(outt_v.at[buf],
                             out_hbm.at[j, :, pl.ds(bcol, BBLK)],
                             osem.at[buf])
            return carry

        lax.fori_loop(0, n_chunks, chunk_body, 0)

        for j in range(n_chunks - NBUF, n_chunks):
            buf = j % NBUF
            pltpu.make_async_copy(
                outt_v.at[buf],
                out_hbm.at[j, :, pl.ds(bcol, BBLK)],
                osem.at[buf]).wait()

    return k


def kernel(input_ids, table, gamma, beta):
    B, L = input_ids.shape
    idsT = input_ids.T.astype(jnp.int32)
    table2 = table.reshape(table.shape[0] // 2, 2 * D)
    out = _make_sc_kernel(B, L)(idsT, table2, gamma, beta)
    return jnp.transpose(out, (2, 0, 1))
